# baseline (speedup 1.0000x reference)
import functools

import jax
import jax.numpy as jnp
from jax import lax
from jax.experimental import pallas as pl
from jax.experimental.pallas import tpu as pltpu
from jax.experimental.pallas import tpu_sc as plsc

N_ENV = 128
N_SRV = 2048
LANES = 16
ENV_CHUNKS = N_ENV // LANES


def _sc_route_body(idx_hbm, cpu_hbm, ram_hbm, acpu_hbm, ccpu_hbm, aram_hbm,
                   cram_hbm, heu_hbm, idx_v, acpu_v, ccpu_v, aram_v, cram_v,
                   cpu_v, ram_v, heu_v):
    cid = lax.axis_index("c")
    sid = lax.axis_index("s")
    wid = sid * 2 + cid

    @pl.when(wid < ENV_CHUNKS)
    def _():
        base = wid * LANES
        pltpu.sync_copy(idx_hbm, idx_v)
        pltpu.sync_copy(acpu_hbm, acpu_v)
        pltpu.sync_copy(ccpu_hbm, ccpu_v)
        pltpu.sync_copy(aram_hbm, aram_v)
        pltpu.sync_copy(cram_hbm, cram_v)
        pltpu.sync_copy(cpu_hbm.at[pl.ds(base, LANES)], cpu_v)
        pltpu.sync_copy(ram_hbm.at[pl.ds(base, LANES)], ram_v)

        lane = jnp.arange(LANES, dtype=jnp.int32)
        rows2 = 2 * base + 2 * lane
        i0 = plsc.load_gather(idx_v, [rows2])
        i1 = plsc.load_gather(idx_v, [rows2 + 1])

        cpu_req = cpu_v[...]
        ram_req = ram_v[...]
        lb0 = ((plsc.load_gather(acpu_v, [i0]) - cpu_req)
               / plsc.load_gather(ccpu_v, [i0])
               + (plsc.load_gather(aram_v, [i0]) - ram_req)
               / plsc.load_gather(cram_v, [i0]))
        lb1 = ((plsc.load_gather(acpu_v, [i1]) - cpu_req)
               / plsc.load_gather(ccpu_v, [i1])
               + (plsc.load_gather(aram_v, [i1]) - ram_req)
               / plsc.load_gather(cram_v, [i1]))
        win1 = lb1 > lb0

        head = idx_v[pl.ds(0, LANES)]
        neg = jnp.full((LANES,), -1, jnp.int32)
        c0 = jnp.max(jnp.where(lane == 0, head, neg))
        c1 = jnp.max(jnp.where(lane == 2, head, neg))
        cand0 = jnp.full((LANES,), c0, jnp.int32)
        cand1 = jnp.full((LANES,), c1, jnp.int32)
        heu_v[...] = jnp.where(win1, cand1, cand0)
        pltpu.sync_copy(heu_v, heu_hbm.at[pl.ds(base, LANES)])


@functools.partial(jax.jit, static_argnames=())
def _sc_route(idx, cpu_req, ram_req, acpu, ccpu, aram, cram):
    mesh = plsc.VectorSubcoreMesh(core_axis_name="c", subcore_axis_name="s")
    return pl.kernel(
        _sc_route_body,
        out_type=jax.ShapeDtypeStruct((N_ENV,), jnp.int32),
        mesh=mesh,
        compiler_params=pltpu.CompilerParams(needs_layout_passes=False),
        scratch_types=[
            pltpu.VMEM((N_ENV * 2,), jnp.int32),
            pltpu.VMEM((N_SRV,), jnp.float32),
            pltpu.VMEM((N_SRV,), jnp.float32),
            pltpu.VMEM((N_SRV,), jnp.float32),
            pltpu.VMEM((N_SRV,), jnp.float32),
            pltpu.VMEM((LANES,), jnp.float32),
            pltpu.VMEM((LANES,), jnp.float32),
            pltpu.VMEM((LANES,), jnp.int32),
        ],
    )(idx, cpu_req, ram_req, acpu, ccpu, aram, cram)


ROWS_PER_BLK = 16
GRID = N_ENV // ROWS_PER_BLK


def _tc_apply_body(x_ref, heu_ref, o_ref):
    x = x_ref[...]
    mx = jnp.max(x, axis=1, keepdims=True)
    cols = lax.broadcasted_iota(jnp.int32, x.shape, 1)
    o_ref[...] = jnp.where(cols == heu_ref[...], mx, x)


def _tc_apply(x, heu2d):
    return pl.pallas_call(
        _tc_apply_body,
        grid=(GRID,),
        in_specs=[
            pl.BlockSpec((ROWS_PER_BLK, N_SRV), lambda i: (i, 0)),
            pl.BlockSpec((ROWS_PER_BLK, 1), lambda i: (i, 0)),
        ],
        out_specs=pl.BlockSpec((ROWS_PER_BLK, N_SRV), lambda i: (i, 0)),
        out_shape=jax.ShapeDtypeStruct((N_ENV, N_SRV), jnp.float32),
    )(x, heu2d)


def kernel(x, cur_vnf_cpu_req, cur_vnf_ram_req, availCPU, CPUcap, availRAM,
           RAMcap, sampled_indexes):
    idx = sampled_indexes.astype(jnp.int32).reshape(-1)
    heu = _sc_route(idx, cur_vnf_cpu_req, cur_vnf_ram_req, availCPU, CPUcap,
                    availRAM, RAMcap)
    return _tc_apply(x, heu.reshape(N_ENV, 1))


# speedup vs baseline: 1.1001x; 1.1001x over previous
import functools

import jax
import jax.numpy as jnp
from jax import lax
from jax.experimental import pallas as pl
from jax.experimental.pallas import tpu as pltpu
from jax.experimental.pallas import tpu_sc as plsc

N_ENV = 128
N_SRV = 2048
LANES = 16
N_WORKERS = 32
ROWS_PER_W = N_ENV // N_WORKERS
FLAT_PER_W = ROWS_PER_W * N_SRV


def _sc_body(x_hbm, idx_hbm, cpu_hbm, ram_hbm, acpu_hbm, ccpu_hbm, aram_hbm,
             cram_hbm, out_hbm, xv, idxv, cpuv, ramv, gb0, gb1,
             a0v, c0v, r0v, d0v, a1v, c1v, r1v, d1v, sem_x, sem_g):
    wid = lax.axis_index("s") * 2 + lax.axis_index("c")
    fbase = wid * FLAT_PER_W

    cp_x = pltpu.async_copy(x_hbm.at[pl.ds(fbase, FLAT_PER_W)], xv, sem_x)

    pltpu.sync_copy(idx_hbm, idxv)
    pltpu.sync_copy(cpu_hbm, cpuv)
    pltpu.sync_copy(ram_hbm, ramv)

    lane = jnp.arange(LANES, dtype=jnp.int32)
    row = jnp.minimum(lane, ROWS_PER_W - 1)
    env2 = 2 * (ROWS_PER_W * wid) + 2 * row
    i0 = plsc.load_gather(idxv, [env2])
    i1 = plsc.load_gather(idxv, [env2 + 1])
    gb0[...] = i0
    gb1[...] = i1

    d = [pltpu.async_copy(acpu_hbm.at[gb0], a0v, sem_g),
         pltpu.async_copy(ccpu_hbm.at[gb0], c0v, sem_g),
         pltpu.async_copy(aram_hbm.at[gb0], r0v, sem_g),
         pltpu.async_copy(cram_hbm.at[gb0], d0v, sem_g),
         pltpu.async_copy(acpu_hbm.at[gb1], a1v, sem_g),
         pltpu.async_copy(ccpu_hbm.at[gb1], c1v, sem_g),
         pltpu.async_copy(aram_hbm.at[gb1], r1v, sem_g),
         pltpu.async_copy(cram_hbm.at[gb1], d1v, sem_g)]
    for cp in d:
        cp.wait()

    creq = plsc.load_gather(cpuv, [ROWS_PER_W * wid + row])
    rreq = plsc.load_gather(ramv, [ROWS_PER_W * wid + row])
    lb0 = (a0v[...] - creq) / c0v[...] + (r0v[...] - rreq) / d0v[...]
    lb1 = (a1v[...] - creq) / c1v[...] + (r1v[...] - rreq) / d1v[...]
    win1 = lb1 > lb0

    head = idxv[pl.ds(0, LANES)]
    neg = jnp.full((LANES,), -1, jnp.int32)
    cand0 = jnp.full((LANES,), jnp.max(jnp.where(lane == 0, head, neg)))
    cand1 = jnp.full((LANES,), jnp.max(jnp.where(lane == 2, head, neg)))
    heu = jnp.where(win1, cand1, cand0)

    cp_x.wait()

    ninf = jnp.full((LANES,), -jnp.inf, jnp.float32)

    def _max_body(off, ms):
        return tuple(
            jnp.maximum(m, xv[pl.ds(r * N_SRV + off, LANES)])
            for r, m in enumerate(ms))

    maxes = plsc.parallel_loop(
        0, N_SRV, LANES, unroll=8,
        carry=(ninf, ninf, ninf, ninf))(_max_body)

    mx = jnp.where(lane == 0, jnp.max(maxes[0]),
                   jnp.where(lane == 1, jnp.max(maxes[1]),
                             jnp.where(lane == 2, jnp.max(maxes[2]),
                                       jnp.max(maxes[3]))))

    pos = row * N_SRV + heu
    plsc.store_scatter(xv, [pos], mx, mask=lane < ROWS_PER_W)

    pltpu.sync_copy(xv, out_hbm.at[pl.ds(fbase, FLAT_PER_W)])


@jax.jit
def _run(x_flat, idx_flat, cpu_req, ram_req, acpu, ccpu, aram, cram):
    mesh = plsc.VectorSubcoreMesh(core_axis_name="c", subcore_axis_name="s")
    return pl.kernel(
        _sc_body,
        out_type=jax.ShapeDtypeStruct((N_ENV * N_SRV,), jnp.float32),
        mesh=mesh,
        compiler_params=pltpu.CompilerParams(needs_layout_passes=False),
        scratch_types=[
            pltpu.VMEM((FLAT_PER_W,), jnp.float32),
            pltpu.VMEM((N_ENV * 2,), jnp.int32),
            pltpu.VMEM((N_ENV,), jnp.float32),
            pltpu.VMEM((N_ENV,), jnp.float32),
            pltpu.VMEM((LANES,), jnp.int32),
            pltpu.VMEM((LANES,), jnp.int32),
            pltpu.VMEM((LANES,), jnp.float32),
            pltpu.VMEM((LANES,), jnp.float32),
            pltpu.VMEM((LANES,), jnp.float32),
            pltpu.VMEM((LANES,), jnp.float32),
            pltpu.VMEM((LANES,), jnp.float32),
            pltpu.VMEM((LANES,), jnp.float32),
            pltpu.VMEM((LANES,), jnp.float32),
            pltpu.VMEM((LANES,), jnp.float32),
            pltpu.SemaphoreType.DMA,
            pltpu.SemaphoreType.DMA,
        ],
    )(x_flat, idx_flat, cpu_req, ram_req, acpu, ccpu, aram, cram)


def kernel(x, cur_vnf_cpu_req, cur_vnf_ram_req, availCPU, CPUcap, availRAM,
           RAMcap, sampled_indexes):
    idx = sampled_indexes.astype(jnp.int32).reshape(-1)
    out = _run(x.reshape(-1), idx, cur_vnf_cpu_req, cur_vnf_ram_req,
               availCPU, CPUcap, availRAM, RAMcap)
    return out.reshape(N_ENV, N_SRV)


# speedup vs baseline: 1.2230x; 1.1118x over previous
import jax
import jax.numpy as jnp
from jax import lax
from jax.experimental import pallas as pl
from jax.experimental.pallas import tpu as pltpu
from jax.experimental.pallas import tpu_sc as plsc

N_ENV = 128
N_SRV = 2048
LANES = 16
N_WORKERS = 16
ROWS_PER_W = N_ENV // N_WORKERS
FLAT_PER_W = ROWS_PER_W * N_SRV


def _sc_body(x_hbm, idx_hbm, cpu_hbm, ram_hbm, acpu_hbm, ccpu_hbm, aram_hbm,
             cram_hbm, out_hbm, xv, idxv, cpuv, ramv, gb0, gb1,
             a0v, c0v, r0v, d0v, a1v, c1v, r1v, d1v, sem_x, sem_g):
    wid = lax.axis_index("s")
    fbase = wid * FLAT_PER_W

    cp_x = pltpu.async_copy(x_hbm.at[pl.ds(fbase, FLAT_PER_W)], xv, sem_x)

    pltpu.sync_copy(idx_hbm, idxv)

    lane = jnp.arange(LANES, dtype=jnp.int32)
    row = jnp.minimum(lane, ROWS_PER_W - 1)
    env2 = 2 * (ROWS_PER_W * wid) + 2 * row
    i0 = plsc.load_gather(idxv, [env2])
    i1 = plsc.load_gather(idxv, [env2 + 1])
    gb0[...] = i0
    gb1[...] = i1

    d = [pltpu.async_copy(acpu_hbm.at[gb0], a0v, sem_g),
         pltpu.async_copy(ccpu_hbm.at[gb0], c0v, sem_g),
         pltpu.async_copy(aram_hbm.at[gb0], r0v, sem_g),
         pltpu.async_copy(cram_hbm.at[gb0], d0v, sem_g),
         pltpu.async_copy(acpu_hbm.at[gb1], a1v, sem_g),
         pltpu.async_copy(ccpu_hbm.at[gb1], c1v, sem_g),
         pltpu.async_copy(aram_hbm.at[gb1], r1v, sem_g),
         pltpu.async_copy(cram_hbm.at[gb1], d1v, sem_g),
         pltpu.async_copy(cpu_hbm, cpuv, sem_g),
         pltpu.async_copy(ram_hbm, ramv, sem_g)]
    for cp in d:
        cp.wait()

    creq = plsc.load_gather(cpuv, [ROWS_PER_W * wid + row])
    rreq = plsc.load_gather(ramv, [ROWS_PER_W * wid + row])
    lb0 = (a0v[...] - creq) / c0v[...] + (r0v[...] - rreq) / d0v[...]
    lb1 = (a1v[...] - creq) / c1v[...] + (r1v[...] - rreq) / d1v[...]
    win1 = lb1 > lb0

    head = idxv[pl.ds(0, LANES)]
    neg = jnp.full((LANES,), -1, jnp.int32)
    cand0 = jnp.full((LANES,), jnp.max(jnp.where(lane == 0, head, neg)))
    cand1 = jnp.full((LANES,), jnp.max(jnp.where(lane == 2, head, neg)))
    heu = jnp.where(win1, cand1, cand0)

    cp_x.wait()

    ninf = jnp.full((LANES,), -jnp.inf, jnp.float32)

    def _max_body(off, ms):
        return tuple(
            jnp.maximum(m, xv[pl.ds(r * N_SRV + off, LANES)])
            for r, m in enumerate(ms))

    maxes = plsc.parallel_loop(
        0, N_SRV, LANES, unroll=4,
        carry=(ninf,) * ROWS_PER_W)(_max_body)

    mx = jnp.full((LANES,), jnp.max(maxes[0]))
    for r in range(1, ROWS_PER_W):
        mx = jnp.where(lane == r, jnp.max(maxes[r]), mx)

    pos = row * N_SRV + heu
    plsc.store_scatter(xv, [pos], mx, mask=lane < ROWS_PER_W)

    pltpu.sync_copy(xv, out_hbm.at[pl.ds(fbase, FLAT_PER_W)])


@jax.jit
def _run(x_flat, idx_flat, cpu_req, ram_req, acpu, ccpu, aram, cram):
    mesh = plsc.VectorSubcoreMesh(core_axis_name="c", subcore_axis_name="s",
                                  num_cores=1)
    return pl.kernel(
        _sc_body,
        out_type=jax.ShapeDtypeStruct((N_ENV * N_SRV,), jnp.float32),
        mesh=mesh,
        compiler_params=pltpu.CompilerParams(needs_layout_passes=False),
        scratch_types=[
            pltpu.VMEM((FLAT_PER_W,), jnp.float32),
            pltpu.VMEM((N_ENV * 2,), jnp.int32),
            pltpu.VMEM((N_ENV,), jnp.float32),
            pltpu.VMEM((N_ENV,), jnp.float32),
            pltpu.VMEM((LANES,), jnp.int32),
            pltpu.VMEM((LANES,), jnp.int32),
            pltpu.VMEM((LANES,), jnp.float32),
            pltpu.VMEM((LANES,), jnp.float32),
            pltpu.VMEM((LANES,), jnp.float32),
            pltpu.VMEM((LANES,), jnp.float32),
            pltpu.VMEM((LANES,), jnp.float32),
            pltpu.VMEM((LANES,), jnp.float32),
            pltpu.VMEM((LANES,), jnp.float32),
            pltpu.VMEM((LANES,), jnp.float32),
            pltpu.SemaphoreType.DMA,
            pltpu.SemaphoreType.DMA,
        ],
    )(x_flat, idx_flat, cpu_req, ram_req, acpu, ccpu, aram, cram)


def kernel(x, cur_vnf_cpu_req, cur_vnf_ram_req, availCPU, CPUcap, availRAM,
           RAMcap, sampled_indexes):
    idx = sampled_indexes.astype(jnp.int32).reshape(-1)
    out = _run(x.reshape(-1), idx, cur_vnf_cpu_req, cur_vnf_ram_req,
               availCPU, CPUcap, availRAM, RAMcap)
    return out.reshape(N_ENV, N_SRV)
